# baseline (speedup 1.0000x reference)
import jax
import jax.numpy as jnp
from jax import lax
from jax.experimental import pallas as pl
from jax.experimental.pallas import tpu as pltpu
from jax.experimental.pallas import tpu_sc as plsc

_NC = 2
_NS = 16
_NW = _NC * _NS

_B = 4096
_L = 200
_D = 64
_RPW = _B // _NW
_T = _RPW * _L
_G = 2
_CT = _G * _RPW
_NCHUNK = _L // _G
_NPOS = 208
_MC = 256


def _body(x_hbm, tok_hbm, pos_hbm, out_hbm, mask_hbm,
          idx_all, posblk, gbufs, tbufs, gidxs, pidxs, carry_v, pos_rare,
          sem_g, sem_w, sem_m, sem_p):
  wid = lax.axis_index("s") * _NC + lax.axis_index("c")
  base = wid * _T

  iota = lax.iota(jnp.int32, 16)
  zeros = jnp.full((16,), 0, jnp.int32)
  ones = jnp.full((16,), 1, jnp.int32)
  iota200 = iota * 200

  pltpu.sync_copy(x_hbm.at[pl.ds(base, _T)], idx_all)
  pltpu.sync_copy(pos_hbm.at[pl.ds(0, _NPOS)], posblk)
  for g in range(_RPW // 16):
    carry_v[pl.ds(g * 16, 16)] = zeros

  def mask_wait(b):
    pltpu.make_async_copy(gidxs[b], mask_hbm.at[pl.ds(0, _MC)],
                          sem_m[b]).wait()

  def mask_chunk(i, b):
    for j in range(_MC // 16):
      v = idx_all[pl.ds(i * _MC + j * 16, 16)]
      gidxs[b][pl.ds(j * 16, 16)] = ones - jnp.minimum(v, ones)
    pltpu.async_copy(gidxs[b], mask_hbm.at[pl.ds(base + i * _MC, _MC)],
                     sem_m[b])

  mask_chunk(jnp.int32(0), 0)
  mask_chunk(jnp.int32(1), 1)

  def mask_body(i, _):
    for b in range(2):
      mask_wait(b)
      mask_chunk(2 * i + b, b)
    return 0

  lax.fori_loop(1, _T // _MC // 2, mask_body, 0)
  mask_wait(0)
  mask_wait(1)

  def stage_a(c, gi):
    l0 = c * _G
    for l_off in range(_G):
      for g in range(_RPW // 16):
        ivec = iota200 + (g * 16 * 200 + l0 + l_off)
        v = plsc.load_gather(idx_all, [ivec])
        nz = jnp.minimum(v, ones)
        cv = carry_v[pl.ds(g * 16, 16)] + nz
        carry_v[pl.ds(g * 16, 16)] = cv
        gidxs[gi][pl.ds(l_off * _RPW + g * 16, 16)] = v
        pidxs[gi][pl.ds(l_off * _RPW + g * 16, 16)] = cv * nz
    mc = jnp.int32(0x7FFFFFFF)
    for g in range(_RPW // 16):
      mc = jnp.minimum(mc, jnp.min(carry_v[pl.ds(g * 16, 16)]))
    return jnp.where(mc == l0 + _G, jnp.int32(1), jnp.int32(0))

  def start_gather(b):
    pltpu.async_copy(tok_hbm.at[gidxs[b]], gbufs[b], sem_g[b])

  def wait_gather(b):
    pltpu.make_async_copy(tok_hbm.at[gidxs[b]], gbufs[b], sem_g[b]).wait()

  def wait_wb(t):
    pltpu.make_async_copy(tbufs[t], out_hbm.at[pl.ds(0, _G * 8), wid],
                          sem_w[t]).wait()

  def combine(c, b, t, flag):
    l0 = c * _G
    gb = gbufs[b]
    tb = tbufs[t]

    def add_tiles(pos_row16):
      def d_body(d, _):
        dt = d // 8
        dr = d - dt * 8
        dvec = zeros + d
        for l_off in range(_G):
          for g in range(_RPW // 16):
            rows = iota + (l_off * _RPW + g * 16)
            tv = plsc.load_gather(gb, [rows, dvec])
            pv = pos_row16(l_off, dvec, rows)
            tb[l_off * 8 + dt, dr, pl.ds(g * 16, 16)] = tv + pv
        return 0
      lax.fori_loop(0, _D, d_body, 0)

    @pl.when(flag == 1)
    def _common():
      def pos_common(l_off, dvec, rows):
        return plsc.load_gather(posblk, [zeros + (l0 + l_off + 1), dvec])
      add_tiles(pos_common)

    @pl.when(flag == 0)
    def _rare():
      pltpu.async_copy(pos_hbm.at[pidxs[b]], pos_rare, sem_p).wait()
      def pos_rare_fn(l_off, dvec, rows):
        return plsc.load_gather(pos_rare, [rows, dvec])
      add_tiles(pos_rare_fn)

    pltpu.async_copy(tb, out_hbm.at[pl.ds(c * _G * 8, _G * 8), wid], sem_w[t])

  def pair(k, flag0, first, last):
    c0 = 2 * k
    c1 = c0 + 1
    flag1 = stage_a(c1, 1)
    start_gather(1)
    if not first:
      wait_wb(0)
    wait_gather(0)
    combine(c0, 0, 0, flag0)
    if not last:
      flag2 = stage_a(c0 + 2, 0)
      start_gather(0)
    else:
      flag2 = jnp.int32(0)
    if not first:
      wait_wb(1)
    wait_gather(1)
    combine(c1, 1, 1, flag1)
    return flag2

  flag0 = stage_a(jnp.int32(0), 0)
  start_gather(0)
  flag2 = pair(jnp.int32(0), flag0, True, False)
  flag2 = lax.fori_loop(
      1, _NCHUNK // 2 - 1, lambda k, f: pair(k, f, False, False), flag2)
  pair(jnp.int32(_NCHUNK // 2 - 1), flag2, False, True)
  wait_wb(0)
  wait_wb(1)


@jax.jit
def _combined_embedding(x_flat, tok_table, pos_table):
  mesh = plsc.VectorSubcoreMesh(
      core_axis_name="c", subcore_axis_name="s",
      num_cores=_NC, num_subcores=_NS)
  out4, mask = pl.kernel(
      _body,
      out_type=(
          jax.ShapeDtypeStruct((_L * 8, _NW, 8, 128), jnp.float32),
          jax.ShapeDtypeStruct((_B * _L,), jnp.int32),
      ),
      mesh=mesh,
      compiler_params=pltpu.CompilerParams(
          use_tc_tiling_on_sc=False, needs_layout_passes=False),
      scratch_types=(
          pltpu.VMEM((_T,), jnp.int32),
          pltpu.VMEM((_NPOS, _D), jnp.float32),
          [pltpu.VMEM((_CT, _D), jnp.float32)] * 2,
          [pltpu.VMEM((_G * 8, 8, 128), jnp.float32)] * 2,
          [pltpu.VMEM((_CT,), jnp.int32)] * 2,
          [pltpu.VMEM((_CT,), jnp.int32)] * 2,
          pltpu.VMEM((_RPW,), jnp.int32),
          pltpu.VMEM((_CT, _D), jnp.float32),
          [pltpu.SemaphoreType.DMA] * 2,
          [pltpu.SemaphoreType.DMA] * 2,
          [pltpu.SemaphoreType.DMA] * 2,
          pltpu.SemaphoreType.DMA,
      ),
  )(x_flat, tok_table, pos_table)
  return out4, mask


def kernel(x, tok_table, pos_table):
  x_flat = x.astype(jnp.int32).reshape(-1)
  out4, mask = _combined_embedding(x_flat, tok_table, pos_table)
  r5 = out4.reshape(_L, 8, _NW, 8, 128)
  out = r5.transpose(2, 4, 0, 1, 3).reshape(_B, _L, _D)
  return out, mask.reshape(_B, _L) != 0


# speedup vs baseline: 2.3456x; 2.3456x over previous
import jax
import jax.numpy as jnp
from jax import lax
from jax.experimental import pallas as pl
from jax.experimental.pallas import tpu as pltpu
from jax.experimental.pallas import tpu_sc as plsc

_NC = 2
_NS = 16
_NW = _NC * _NS

_B = 4096
_L = 200
_D = 64
_T = (_B * _L) // _NW
_C = 400
_NCHUNK = _T // _C
_NBUF = 3
_NPOS = 208


def _body(x_hbm, tok_hbm, pos_hbm, out_hbm, mask_hbm,
          idx_all, posblk, tok_bufs, mask_bufs, pidx_v, pos_rare,
          sem_t, sem_o, sem_m, sem_p):
  wid = lax.axis_index("s") * _NC + lax.axis_index("c")
  base = wid * _T

  ones = jnp.full((16,), 1, jnp.int32)
  zeros = jnp.full((16,), 0, jnp.int32)
  eights = jnp.full((16,), 8, jnp.int32)
  fh = jnp.minimum(jnp.maximum(eights - lax.iota(jnp.int32, 16), zeros), ones)
  sh = ones - fh

  pltpu.sync_copy(x_hbm.at[pl.ds(base, _T)], idx_all)
  pltpu.sync_copy(pos_hbm.at[pl.ds(0, _NPOS)], posblk)

  def start_gather(c, b):
    sl = pl.ds(c * _C, _C)
    pltpu.async_copy(tok_hbm.at[idx_all.at[sl]], tok_bufs[b], sem_t[b])

  def wait_gather(c, b):
    pltpu.make_async_copy(tok_hbm.at[idx_all.at[pl.ds(c * _C, _C)]],
                          tok_bufs[b], sem_t[b]).wait()

  def wait_wb(b):
    pltpu.make_async_copy(tok_bufs[b], out_hbm.at[pl.ds(0, _C)],
                          sem_o[b]).wait()
    pltpu.make_async_copy(mask_bufs[b], mask_hbm.at[pl.ds(0, _C)],
                          sem_m[b]).wait()

  def process(c, b):
    off = c * _C
    tok_v = tok_bufs[b]
    mask_v = mask_bufs[b]

    minv = jnp.int32(1)
    for j in range(_C // 16):
      v = idx_all[pl.ds(off + j * 16, 16)]
      nz = jnp.minimum(v, ones)
      mask_v[pl.ds(j * 16, 16)] = ones - nz
      minv = jnp.minimum(minv, jnp.min(nz))

    wait_gather(c, b)

    @pl.when(minv > 0)
    def _common():
      def add_body(r, _):
        for k in range(_D // 16):
          sl = pl.ds(k * 16, 16)
          p = posblk[r + 1, sl]
          tok_v[r, sl] = tok_v[r, sl] + p
          tok_v[r + _L, sl] = tok_v[r + _L, sl] + p
        return 0
      lax.fori_loop(0, _L, add_body, 0)

    @pl.when(minv == 0)
    def _rare():
      carry = jnp.int32(0)
      for j in range(_C // 16):
        v = idx_all[pl.ds(off + j * 16, 16)]
        nz = jnp.minimum(v, ones)
        cs = plsc.cumsum(nz)
        if j == 12:
          s7 = jnp.sum(nz * fh)
          pos = (cs + carry * fh - s7 * sh) * nz
          carry = jnp.sum(nz * sh)
        else:
          pos = (cs + carry) * nz
          carry = carry + jnp.sum(nz)
        pidx_v[pl.ds(j * 16, 16)] = pos

      for h in range(2):
        pltpu.async_copy(pos_hbm.at[pidx_v.at[pl.ds(h * _L, _L)]],
                         pos_rare, sem_p).wait()

        def add_body(r, _):
          for k in range(_D // 16):
            sl = pl.ds(k * 16, 16)
            tok_v[h * _L + r, sl] = tok_v[h * _L + r, sl] + pos_rare[r, sl]
          return 0
        lax.fori_loop(0, _L, add_body, 0)

    pltpu.async_copy(tok_v, out_hbm.at[pl.ds(base + off, _C)], sem_o[b])
    pltpu.async_copy(mask_v, mask_hbm.at[pl.ds(base + off, _C)], sem_m[b])

  start_gather(0, 0)
  start_gather(1, 1)

  def position(c, b, first_prefetch=False, guard_prefetch=False):
    process(c, b)
    nb = (b + 2) % _NBUF

    def prefetch():
      if not first_prefetch:
        wait_wb(nb)
      start_gather(c + 2, nb)

    if guard_prefetch:
      @pl.when(c + 2 < _NCHUNK)
      def _():
        prefetch()
    else:
      prefetch()

  position(jnp.int32(0), 0, first_prefetch=True)
  position(jnp.int32(1), 1)
  position(jnp.int32(2), 2)

  def tri(k, _):
    c0 = 3 * k
    position(c0, 0, guard_prefetch=True)
    position(c0 + 1, 1, guard_prefetch=True)
    position(c0 + 2, 2, guard_prefetch=True)
    return 0

  lax.fori_loop(1, _NCHUNK // 3, tri, 0)

  process(jnp.int32(_NCHUNK - 1), 0)
  for b in range(_NBUF):
    wait_wb(b)


@jax.jit
def _combined_embedding(x_flat, tok_table, pos_table):
  mesh = plsc.VectorSubcoreMesh(
      core_axis_name="c", subcore_axis_name="s",
      num_cores=_NC, num_subcores=_NS)
  out, mask = pl.kernel(
      _body,
      out_type=(
          jax.ShapeDtypeStruct((_B * _L, _D), jnp.float32),
          jax.ShapeDtypeStruct((_B * _L,), jnp.int32),
      ),
      mesh=mesh,
      compiler_params=pltpu.CompilerParams(
          use_tc_tiling_on_sc=False, needs_layout_passes=False),
      scratch_types=(
          pltpu.VMEM((_T,), jnp.int32),
          pltpu.VMEM((_NPOS, _D), jnp.float32),
          [pltpu.VMEM((_C, _D), jnp.float32)] * _NBUF,
          [pltpu.VMEM((_C,), jnp.int32)] * _NBUF,
          pltpu.VMEM((_C,), jnp.int32),
          pltpu.VMEM((_L, _D), jnp.float32),
          [pltpu.SemaphoreType.DMA] * _NBUF,
          [pltpu.SemaphoreType.DMA] * _NBUF,
          [pltpu.SemaphoreType.DMA] * _NBUF,
          pltpu.SemaphoreType.DMA,
      ),
  )(x_flat, tok_table, pos_table)
  return out, mask


def kernel(x, tok_table, pos_table):
  x_flat = x.astype(jnp.int32).reshape(-1)
  out, mask = _combined_embedding(x_flat, tok_table, pos_table)
  return out.reshape(_B, _L, _D), mask.reshape(_B, _L) != 0
